# baseline (speedup 1.0000x reference)
import functools

import jax
import jax.numpy as jnp
from jax import lax
from jax.experimental import pallas as pl
from jax.experimental.pallas import tpu as pltpu
from jax.experimental.pallas import tpu_sc as plsc

N_SEGMENT = 1000
D_MODEL = 1024
N_TOKENS = 4 * 8192

_NC = 2
_NS = 16
_NW = _NC * _NS
_TOK_PER_W = N_TOKENS // _NW
_CHUNK = 32
_NCHUNK = _TOK_PER_W // _CHUNK
_TBL = N_SEGMENT * D_MODEL
_STAGE = _TBL // _NS

_mesh = plsc.VectorSubcoreMesh(core_axis_name="c", subcore_axis_name="s")


@functools.partial(
    pl.kernel,
    mesh=_mesh,
    out_type=jax.ShapeDtypeStruct((N_TOKENS * D_MODEL,), jnp.float32),
    scratch_types=[
        pltpu.VMEM((_TOK_PER_W,), jnp.int32),
        pltpu.VMEM((2 * _CHUNK * D_MODEL,), jnp.float32),
        pltpu.VMEM_SHARED((_TBL,), jnp.float32),
        pltpu.SemaphoreType.DMA,
        pltpu.SemaphoreType.DMA,
        pltpu.SemaphoreType.DMA,
        pltpu.SemaphoreType.DMA,
    ],
)
def _emb_lookup(x_hbm, w_hbm, out_hbm, idx_v, rows_v, w_sh,
                gsem0, gsem1, ssem0, ssem1):
    sid = lax.axis_index("s")
    wid = sid * _NC + lax.axis_index("c")
    base = wid * _TOK_PER_W

    soff = pl.multiple_of(sid * _STAGE, 8)
    pltpu.sync_copy(w_hbm.at[pl.ds(soff, _STAGE)],
                    w_sh.at[pl.ds(soff, _STAGE)])

    pltpu.sync_copy(x_hbm.at[pl.ds(base, _TOK_PER_W)], idx_v)

    plsc.subcore_barrier()

    gsem = (gsem0, gsem1)
    ssem = (ssem0, ssem1)
    lane = lax.iota(jnp.int32, 16)

    def fill(g, slot):
        hs = []
        for kk in range(_CHUNK // 16):
            vec = idx_v[pl.ds((g * _CHUNK + kk * 16), 16)]
            for j in range(16):
                row = lax.squeeze(lax.slice(vec, (j,), (j + 1,)), (0,))
                roff = pl.multiple_of(row * D_MODEL, 8)
                dst = (slot * _CHUNK + kk * 16 + j) * D_MODEL
                hs.append(pltpu.async_copy(
                    w_sh.at[pl.ds(roff, D_MODEL)],
                    rows_v.at[pl.ds(dst, D_MODEL)],
                    gsem[slot],
                ))
        return hs

    def scatter(g, slot):
        return pltpu.async_copy(
            rows_v.at[pl.ds(slot * _CHUNK * D_MODEL, _CHUNK * D_MODEL)],
            out_hbm.at[pl.ds((base + g * _CHUNK) * D_MODEL,
                             _CHUNK * D_MODEL)],
            ssem[slot],
        )

    def wait_scatter(slot):
        pltpu.make_async_copy(
            rows_v.at[pl.ds(slot * _CHUNK * D_MODEL, _CHUNK * D_MODEL)],
            out_hbm.at[pl.ds(0, _CHUNK * D_MODEL)],
            ssem[slot],
        ).wait()

    def body(k, carry):
        g0 = k * 2
        g1 = g0 + 1

        @pl.when(k > 0)
        def _drain0():
            wait_scatter(0)

        h0 = fill(g0, 0)

        @pl.when(k > 0)
        def _drain1():
            wait_scatter(1)

        h1 = fill(g1, 1)
        for h in h0:
            h.wait()
        scatter(g0, 0)
        for h in h1:
            h.wait()
        scatter(g1, 1)
        return carry

    lax.fori_loop(0, _NCHUNK // 2, body, 0)
    wait_scatter(0)
    wait_scatter(1)


def kernel(x, weight):
    out = _emb_lookup(x.reshape(N_TOKENS), weight.reshape(_TBL))
    return out.reshape(x.shape[0], x.shape[1], D_MODEL)


# speedup vs baseline: 1.9071x; 1.9071x over previous
import functools

import jax
import jax.numpy as jnp
from jax import lax
from jax.experimental import pallas as pl
from jax.experimental.pallas import tpu as pltpu
from jax.experimental.pallas import tpu_sc as plsc

N_SEGMENT = 1000
D_MODEL = 1024
N_TOKENS = 4 * 8192

_NC = 2
_NS = 16
_NW = _NC * _NS
_TOK_PER_W = N_TOKENS // _NW
_CHUNK = 16
_NBUF = 3
_NCHUNK = _TOK_PER_W // _CHUNK
_CD = _CHUNK * D_MODEL

_mesh = plsc.VectorSubcoreMesh(core_axis_name="c", subcore_axis_name="s")


@functools.partial(
    pl.kernel,
    mesh=_mesh,
    out_type=jax.ShapeDtypeStruct((N_TOKENS, D_MODEL), jnp.float32),
    scratch_types=[
        pltpu.VMEM((_TOK_PER_W,), jnp.int32),
        pltpu.VMEM((_NBUF, _CHUNK, D_MODEL), jnp.float32),
        pltpu.VMEM_SHARED((_NS * _NBUF, _CHUNK, D_MODEL), jnp.float32),
    ]
    + [pltpu.SemaphoreType.DMA] * (3 * _NBUF),
)
def _emb_lookup(x_hbm, w_hbm, out_hbm, idx_v, rows_v, sp_ring, *sems):
    sid = lax.axis_index("s")
    wid = sid * _NC + lax.axis_index("c")
    base = wid * _TOK_PER_W
    gsem = sems[:_NBUF]
    csem = sems[_NBUF:2 * _NBUF]
    osem = sems[2 * _NBUF:]

    pltpu.sync_copy(x_hbm.at[pl.ds(base, _TOK_PER_W)], idx_v)

    def gather(g, s):
        return pltpu.async_copy(
            w_hbm.at[idx_v.at[pl.ds(g * _CHUNK, _CHUNK)]],
            rows_v.at[s], gsem[s],
        )

    def to_spmem(s):
        return pltpu.async_copy(
            rows_v.at[s], sp_ring.at[sid * _NBUF + s], csem[s],
        )

    def to_hbm(g, s):
        return pltpu.async_copy(
            sp_ring.at[sid * _NBUF + s],
            out_hbm.at[pl.ds(base + g * _CHUNK, _CHUNK)],
            osem[s],
        )

    gp = [None] * _NBUF
    cp = [None] * _NBUF
    op = [None] * _NBUF
    for t in range(_NCHUNK + 2):
        if t < _NCHUNK:
            s = t % _NBUF
            if op[s] is not None:
                op[s].wait()
                op[s] = None
            gp[s] = gather(t, s)
        if 1 <= t < _NCHUNK + 1:
            c = t - 1
            s = c % _NBUF
            gp[s].wait()
            gp[s] = None
            cp[s] = to_spmem(s)
        if 2 <= t:
            o = t - 2
            s = o % _NBUF
            cp[s].wait()
            cp[s] = None
            op[s] = to_hbm(o, s)
    for p in op:
        if p is not None:
            p.wait()


def kernel(x, weight):
    out = _emb_lookup(x.reshape(N_TOKENS), weight)
    return out.reshape(x.shape[0], x.shape[1], D_MODEL)


# speedup vs baseline: 1.9120x; 1.0025x over previous
import functools

import jax
import jax.numpy as jnp
from jax import lax
from jax.experimental import pallas as pl
from jax.experimental.pallas import tpu as pltpu
from jax.experimental.pallas import tpu_sc as plsc

N_SEGMENT = 1000
D_MODEL = 1024
N_TOKENS = 4 * 8192

_NC = 2
_NS = 16
_NW = _NC * _NS
_TOK_PER_W = N_TOKENS // _NW
_CHUNK = 32
_NBUF = 3
_NCHUNK = _TOK_PER_W // _CHUNK

_mesh = plsc.VectorSubcoreMesh(core_axis_name="c", subcore_axis_name="s")


@functools.partial(
    pl.kernel,
    mesh=_mesh,
    out_type=jax.ShapeDtypeStruct((N_TOKENS, D_MODEL), jnp.float32),
    scratch_types=[
        pltpu.VMEM((_TOK_PER_W,), jnp.int32),
        pltpu.VMEM((_NBUF, _CHUNK, D_MODEL), jnp.float32),
    ]
    + [pltpu.SemaphoreType.DMA] * (2 * _NBUF),
)
def _emb_lookup(x_hbm, w_hbm, out_hbm, idx_v, rows_v, *sems):
    sid = lax.axis_index("s")
    wid = sid * _NC + lax.axis_index("c")
    base = wid * _TOK_PER_W
    gsem = sems[:_NBUF]
    ssem = sems[_NBUF:]

    pltpu.sync_copy(x_hbm.at[pl.ds(base, _TOK_PER_W)], idx_v)

    def gather(g, s):
        return pltpu.async_copy(
            w_hbm.at[idx_v.at[pl.ds(g * _CHUNK, _CHUNK)]],
            rows_v.at[s], gsem[s],
        )

    def scatter(g, s):
        return pltpu.async_copy(
            rows_v.at[s], out_hbm.at[pl.ds(base + g * _CHUNK, _CHUNK)],
            ssem[s],
        )

    gpend = [None] * _NBUF
    spend = [None] * _NBUF
    ahead = _NBUF - 1
    for g in range(min(ahead, _NCHUNK)):
        gpend[g % _NBUF] = gather(g, g % _NBUF)
    for g in range(_NCHUNK):
        s = g % _NBUF
        nxt = g + ahead
        if nxt < _NCHUNK:
            ns = nxt % _NBUF
            if spend[ns] is not None:
                spend[ns].wait()
                spend[ns] = None
            gpend[ns] = gather(nxt, ns)
        gpend[s].wait()
        gpend[s] = None
        spend[s] = scatter(g, s)
    for p in spend:
        if p is not None:
            p.wait()


def kernel(x, weight):
    out = _emb_lookup(x.reshape(N_TOKENS), weight)
    return out.reshape(x.shape[0], x.shape[1], D_MODEL)
